# baseline (speedup 1.0000x reference)
import functools

import jax
import jax.numpy as jnp
from jax import lax
from jax.experimental import pallas as pl
from jax.experimental.pallas import tpu as pltpu
from jax.experimental.pallas import tpu_sc as plsc

HIDDEN = 64
LANES = 16
BLK = 128
NBUF = 4


def _detile_table(emb_t, tail_lin, nc, ns):
    hidden, vocab = emb_t.shape
    nw = nc * ns
    n_full = vocab // BLK
    tail = vocab - n_full * BLK
    assert tail in (0, 64)
    base = n_full // nw
    rem = n_full % nw

    mesh = plsc.VectorSubcoreMesh(core_axis_name="c", subcore_axis_name="s")

    @functools.partial(
        pl.kernel,
        out_type=jax.ShapeDtypeStruct((vocab // 2, 2 * HIDDEN), jnp.float32),
        mesh=mesh,
        compiler_params=pltpu.CompilerParams(use_tc_tiling_on_sc=True,
                                             needs_layout_passes=False),
        scratch_types=[
            pltpu.VMEM((2, HIDDEN, BLK + 1), jnp.float32),
            pltpu.VMEM((2, HIDDEN, BLK), jnp.float32),
            pltpu.SemaphoreType.DMA,
            pltpu.SemaphoreType.DMA,
            pltpu.SemaphoreType.DMA,
            pltpu.SemaphoreType.DMA,
        ],
    )
    def dk(tab_hbm, tail_hbm, out_hbm, in_v, out_v,
           isem0, isem1, osem0, osem1):
        wid = lax.axis_index("s") * nc + lax.axis_index("c")
        isems = (isem0, isem1)
        osems = (osem0, osem1)
        trips = base + jnp.where(wid < rem, 1, 0)
        iota = lax.iota(jnp.int32, LANES)
        hvecs = [m * LANES + iota for m in range(HIDDEN // LANES)]

        def chunk_of(t):
            return wid + nw * t

        def fire_in(c, b):
            pltpu.async_copy(tab_hbm.at[:, pl.ds(c * BLK, BLK)],
                             in_v.at[b, :, pl.ds(0, BLK)], isems[b])

        def wait_in(c, b):
            pltpu.make_async_copy(tab_hbm.at[:, pl.ds(c * BLK, BLK)],
                                  in_v.at[b, :, pl.ds(0, BLK)],
                                  isems[b]).wait()

        def fire_out(c, b):
            pltpu.async_copy(out_v.at[b],
                             out_hbm.at[pl.ds(c * (BLK // 2), BLK // 2)],
                             osems[b])

        def wait_out(c, b):
            pltpu.make_async_copy(out_v.at[b],
                                  out_hbm.at[pl.ds(c * (BLK // 2), BLK // 2)],
                                  osems[b]).wait()

        def transpose(b):
            @plsc.parallel_loop(0, BLK, unroll=2)
            def _(v):
                vvec = jnp.full((LANES,), v, jnp.int32)
                po = (v % 2) * HIDDEN
                pr = v // 2
                for m in range(HIDDEN // LANES):
                    x = plsc.load_gather(in_v.at[b], [hvecs[m], vvec])
                    out_v[b, pr, pl.ds(po + m * LANES, LANES)] = x

        fire_in(chunk_of(0), 0)
        fire_in(chunk_of(1), 1)

        def body(t, carry):
            for b in range(2):
                tb = t + b

                @pl.when(tb < trips)
                def _():
                    c = chunk_of(tb)
                    wait_in(c, b)
                    transpose(b)

                    @pl.when(tb - 2 >= 0)
                    def _():
                        wait_out(chunk_of(tb - 2), b)

                    fire_out(c, b)

                    @pl.when(tb + 2 < trips)
                    def _():
                        fire_in(chunk_of(tb + 2), b)
            return carry

        lax.fori_loop(0, base + 1, lambda t, cr: body(2 * t, cr), 0)

        for b in range(2):
            @pl.when(trips - 2 + b >= 0)
            def _():
                wait_out(chunk_of(trips - 2 + b), b)

        if tail:
            @pl.when(wid == 0)
            def _():
                pltpu.sync_copy(
                    tail_hbm,
                    out_hbm.at[pl.ds(n_full * (BLK // 2), tail // 2)])

    return dk(emb_t, tail_lin)


def kernel(input_id, embedding, pos_code):
    batch, seq = input_id.shape
    info = plsc.get_sparse_core_info()
    nc, ns = info.num_cores, info.num_subcores
    nw = nc * ns
    assert batch == BLK * nw and seq % (2 * NBUF) == 0

    idx4 = (input_id.T.reshape(seq // 8, 8, batch // BLK, BLK)
            .transpose(0, 2, 1, 3))
    vocab = embedding.shape[0]

    mesh = plsc.VectorSubcoreMesh(core_axis_name="c", subcore_axis_name="s")

    @functools.partial(
        pl.kernel,
        out_type=jax.ShapeDtypeStruct((seq, 8, nw, 8, BLK), jnp.float32),
        mesh=mesh,
        compiler_params=pltpu.CompilerParams(use_tc_tiling_on_sc=False,
                                             needs_layout_passes=False),
        scratch_types=[
            pltpu.VMEM((2, NBUF, BLK), jnp.int32),
            pltpu.VMEM((NBUF, BLK, HIDDEN), jnp.float32),
            pltpu.VMEM((NBUF, 8, 8, BLK + 1), jnp.float32),
            pltpu.VMEM((HIDDEN,), jnp.float32),
        ] + [pltpu.SemaphoreType.DMA] * (2 * NBUF),
    )
    def k(idx_hbm, tab_hbm, pc_hbm, out_hbm,
          idx_v, rows_v, dst_v, pos_v, *sems):
        gsems = sems[:NBUF]
        osems = sems[NBUF:]
        wid = lax.axis_index("s") * nc + lax.axis_index("c")
        b0 = wid * BLK

        pltpu.sync_copy(pc_hbm.at[0, seq], pos_v)
        pvecs = [pos_v[pl.ds(j * LANES, LANES)] for j in range(HIDDEN // LANES)]
        iota = lax.iota(jnp.int32, LANES)
        i0s = [2 * j + iota // 8 for j in range(HIDDEN // LANES)]
        i1 = iota % 8

        def load_idx(q, p):
            pltpu.sync_copy(
                idx_hbm.at[q // 2, wid, pl.ds(NBUF * (q % 2), NBUF)],
                idx_v.at[p])

        def fire_gather(b, p):
            pltpu.async_copy(tab_hbm.at[idx_v.at[p, b]],
                             rows_v.at[b], gsems[b])

        def wait_gather(b, p):
            pltpu.make_async_copy(tab_hbm.at[idx_v.at[p, b]],
                                  rows_v.at[b], gsems[b]).wait()

        def transpose_add(b):
            rows = rows_v
            dst = dst_v

            @plsc.parallel_loop(0, BLK, unroll=4)
            def _(r):
                rvec = jnp.full((LANES,), r, jnp.int32)
                for j in range(HIDDEN // LANES):
                    v = rows[b, r, pl.ds(j * LANES, LANES)] + pvecs[j]
                    plsc.store_scatter(dst.at[b], [i0s[j], i1, rvec], v)

        def fire_out(s, b):
            pltpu.async_copy(dst_v.at[b, :, :, pl.ds(0, BLK)],
                             out_hbm.at[s, :, wid], osems[b])

        def wait_out(s, b):
            pltpu.make_async_copy(dst_v.at[b, :, :, pl.ds(0, BLK)],
                                  out_hbm.at[s, :, wid], osems[b]).wait()

        n_quads = seq // NBUF

        load_idx(0, 0)
        for b in range(NBUF):
            fire_gather(b, 0)
        load_idx(1, 1)
        for b in range(NBUF):
            wait_gather(b, 0)
            transpose_add(b)
            fire_out(b, b)
            fire_gather(b, 1)

        def quad_step(q, p, prefetch):
            s0 = NBUF * q
            if prefetch:
                load_idx(q + 1, 1 - p)
            for b in range(NBUF):
                s = s0 + b
                wait_gather(b, p)
                wait_out(s - NBUF, b)
                transpose_add(b)
                fire_out(s, b)
                if prefetch:
                    fire_gather(b, 1 - p)

        def body(i, carry):
            quad_step(1 + 2 * i, 1, True)
            quad_step(2 + 2 * i, 0, True)
            return carry

        lax.fori_loop(0, (n_quads - 2) // 2, body, 0)

        quad_step(n_quads - 1, 1, False)
        for b in range(NBUF):
            wait_out(seq - NBUF + b, b)

    n_full_rows = (vocab // BLK) * BLK
    tail_lin = embedding[n_full_rows:, :].reshape(-1, 2 * HIDDEN)
    tab_lin = _detile_table(embedding.T, tail_lin, nc, ns).reshape(
        vocab, HIDDEN)
    tmp = k(idx4, tab_lin, pos_code)
    return tmp.transpose((2, 4, 0, 1, 3)).reshape(batch, seq, HIDDEN)


# speedup vs baseline: 1.2671x; 1.2671x over previous
import functools

import jax
import jax.numpy as jnp
from jax import lax
from jax.experimental import pallas as pl
from jax.experimental.pallas import tpu as pltpu
from jax.experimental.pallas import tpu_sc as plsc

HIDDEN = 64
LANES = 16
BLK = 128
NBUF = 4


def kernel(input_id, embedding, pos_code):
    batch, seq = input_id.shape
    info = plsc.get_sparse_core_info()
    nc, ns = info.num_cores, info.num_subcores
    nw = nc * ns
    assert batch == BLK * nw and seq % (2 * NBUF) == 0

    idx_t = input_id.T

    mesh = plsc.VectorSubcoreMesh(core_axis_name="c", subcore_axis_name="s")

    @functools.partial(
        pl.kernel,
        out_type=jax.ShapeDtypeStruct((seq, 8, nw, 8, BLK), jnp.float32),
        mesh=mesh,
        compiler_params=pltpu.CompilerParams(use_tc_tiling_on_sc=False,
                                             needs_layout_passes=False),
        scratch_types=[
            pltpu.VMEM((2, NBUF, BLK), jnp.int32),
            pltpu.VMEM((NBUF, BLK, HIDDEN), jnp.float32),
            pltpu.VMEM((NBUF, 8, 8, BLK + 1), jnp.float32),
            pltpu.VMEM((HIDDEN,), jnp.float32),
        ] + [pltpu.SemaphoreType.DMA] * (2 * NBUF),
    )
    def k(idx_hbm, tab_hbm, pc_hbm, out_hbm,
          idx_v, rows_v, dst_v, pos_v, *sems):
        gsems = sems[:NBUF]
        osems = sems[NBUF:]
        wid = lax.axis_index("s") * nc + lax.axis_index("c")
        b0 = wid * BLK

        pltpu.sync_copy(pc_hbm.at[0, seq], pos_v)
        pvecs = [pos_v[pl.ds(j * LANES, LANES)] for j in range(HIDDEN // LANES)]
        iota = lax.iota(jnp.int32, LANES)
        i0s = [2 * j + iota // 8 for j in range(HIDDEN // LANES)]
        i1 = iota % 8

        def load_idx(q, p):
            pltpu.sync_copy(
                idx_hbm.at[pl.ds(NBUF * q, NBUF), pl.ds(b0, BLK)],
                idx_v.at[p])

        def fire_gather(b, p):
            pltpu.async_copy(tab_hbm.at[idx_v.at[p, b]],
                             rows_v.at[b], gsems[b])

        def wait_gather(b, p):
            pltpu.make_async_copy(tab_hbm.at[idx_v.at[p, b]],
                                  rows_v.at[b], gsems[b]).wait()

        def transpose_add(b):
            rows = rows_v
            dst = dst_v

            @plsc.parallel_loop(0, BLK, unroll=4)
            def _(r):
                rvec = jnp.full((LANES,), r, jnp.int32)
                for j in range(HIDDEN // LANES):
                    v = rows[b, r, pl.ds(j * LANES, LANES)] + pvecs[j]
                    plsc.store_scatter(dst.at[b], [i0s[j], i1, rvec], v)

        def fire_out(s, b):
            pltpu.async_copy(dst_v.at[b, :, :, pl.ds(0, BLK)],
                             out_hbm.at[s, :, wid], osems[b])

        def wait_out(s, b):
            pltpu.make_async_copy(dst_v.at[b, :, :, pl.ds(0, BLK)],
                                  out_hbm.at[s, :, wid], osems[b]).wait()

        n_quads = seq // NBUF

        load_idx(0, 0)
        for b in range(NBUF):
            fire_gather(b, 0)
        load_idx(1, 1)
        for b in range(NBUF):
            wait_gather(b, 0)
            transpose_add(b)
            fire_out(b, b)
            fire_gather(b, 1)

        def quad_step(q, p, prefetch):
            s0 = NBUF * q
            if prefetch:
                load_idx(q + 1, 1 - p)
            for b in range(NBUF):
                s = s0 + b
                wait_gather(b, p)
                wait_out(s - NBUF, b)
                transpose_add(b)
                fire_out(s, b)
                if prefetch:
                    fire_gather(b, 1 - p)

        def body(i, carry):
            quad_step(1 + 2 * i, 1, True)
            quad_step(2 + 2 * i, 0, True)
            return carry

        lax.fori_loop(0, (n_quads - 2) // 2, body, 0)

        quad_step(n_quads - 1, 1, False)
        for b in range(NBUF):
            wait_out(seq - NBUF + b, b)

    tmp = k(idx_t, embedding, pos_code)
    return tmp.transpose((2, 4, 0, 1, 3)).reshape(batch, seq, HIDDEN)
